# baseline (speedup 1.0000x reference)
import jax
import jax.numpy as jnp
from jax.experimental import pallas as pl
from jax.experimental.pallas import tpu as pltpu

TOKENS = 32768
D_MODEL = 768
N_EXPERTS = 8
TOP_K = 2

CT = 2048
NS = 1
BT = CT * NS
NCHUNK = TOKENS // CT


def _router_chunk(x, w, p_ref, ew_ref, ei_ref, k):
    lt = jax.lax.dot_general(
        w, x, (((1,), (1,)), ((), ())), preferred_element_type=jnp.float32
    )
    m = jnp.max(lt, axis=0, keepdims=True)
    e = jnp.exp(lt - m)
    p = e / jnp.sum(e, axis=0, keepdims=True)
    p_ref[:, pl.ds(k * CT, CT)] = p

    m1 = p[0:1, :]
    m2 = jnp.full((1, CT), -1.0, jnp.float32)
    i1 = jnp.zeros((1, CT), jnp.int32)
    i2 = jnp.zeros((1, CT), jnp.int32)
    for ei in range(1, N_EXPERTS):
        v = p[ei : ei + 1, :]
        ec = jnp.full((1, CT), ei, jnp.int32)
        gt1 = v > m1
        gt2 = v > m2
        i2 = jnp.where(gt1, i1, jnp.where(gt2, ec, i2))
        m2 = jnp.where(gt1, m1, jnp.where(gt2, v, m2))
        i1 = jnp.where(gt1, ec, i1)
        m1 = jnp.where(gt1, v, m1)
    ew_ref[:, pl.ds(k * CT, CT)] = jnp.concatenate([m1, m2], axis=0)
    ei_ref[:, pl.ds(k * CT, CT)] = jnp.concatenate([i1, i2], axis=0)


def _router_body(*refs):
    x_refs = refs[:NS]
    w_ref, p_ref, ew_ref, ei_ref = refs[NS:]
    w = w_ref[...]
    for k in range(NS):
        _router_chunk(x_refs[k][0], w, p_ref, ew_ref, ei_ref, k)


def kernel(x, W):
    xc = x.reshape(NCHUNK, CT, D_MODEL)

    def chunk_spec(k):
        return pl.BlockSpec((1, CT, D_MODEL), lambda i, k=k: (i * NS + k, 0, 0))

    probs_em, ew_em, ei_em = pl.pallas_call(
        _router_body,
        grid=(TOKENS // BT,),
        in_specs=[chunk_spec(k) for k in range(NS)]
        + [pl.BlockSpec((N_EXPERTS, D_MODEL), lambda i: (0, 0))],
        out_specs=[
            pl.BlockSpec((N_EXPERTS, BT), lambda i: (0, i)),
            pl.BlockSpec((TOP_K, BT), lambda i: (0, i)),
            pl.BlockSpec((TOP_K, BT), lambda i: (0, i)),
        ],
        out_shape=[
            jax.ShapeDtypeStruct((N_EXPERTS, TOKENS), jnp.float32),
            jax.ShapeDtypeStruct((TOP_K, TOKENS), jnp.float32),
            jax.ShapeDtypeStruct((TOP_K, TOKENS), jnp.int32),
        ],
        compiler_params=pltpu.CompilerParams(
            dimension_semantics=("arbitrary",),
        ),
    )(*([xc] * NS), W)
    return (probs_em.T, ew_em.T, ei_em.T)


# speedup vs baseline: 1.0269x; 1.0269x over previous
import jax
import jax.numpy as jnp
from jax.experimental import pallas as pl
from jax.experimental.pallas import tpu as pltpu

TOKENS = 32768
D_MODEL = 768
N_EXPERTS = 8
TOP_K = 2

CT = 2048
NS = 2
BT = CT * NS
NCHUNK = TOKENS // CT


def _router_chunk(x, w, p_ref, ew_ref, ei_ref, k):
    lt = jax.lax.dot_general(
        w, x, (((1,), (1,)), ((), ())), preferred_element_type=jnp.float32
    )
    m = jnp.max(lt, axis=0, keepdims=True)
    e = jnp.exp(lt - m)
    p = e / jnp.sum(e, axis=0, keepdims=True)
    p_ref[:, pl.ds(k * CT, CT)] = p

    m1 = p[0:1, :]
    m2 = jnp.full((1, CT), -1.0, jnp.float32)
    i1 = jnp.zeros((1, CT), jnp.int32)
    i2 = jnp.zeros((1, CT), jnp.int32)
    for ei in range(1, N_EXPERTS):
        v = p[ei : ei + 1, :]
        ec = jnp.full((1, CT), ei, jnp.int32)
        gt1 = v > m1
        gt2 = v > m2
        i2 = jnp.where(gt1, i1, jnp.where(gt2, ec, i2))
        m2 = jnp.where(gt1, m1, jnp.where(gt2, v, m2))
        i1 = jnp.where(gt1, ec, i1)
        m1 = jnp.where(gt1, v, m1)
    ew_ref[:, pl.ds(k * CT, CT)] = jnp.concatenate([m1, m2], axis=0)
    ei_ref[:, pl.ds(k * CT, CT)] = jnp.concatenate([i1, i2], axis=0)


def _router_body(*refs):
    x_refs = refs[:NS]
    w_ref, p_ref, ew_ref, ei_ref = refs[NS:]
    w = w_ref[...]
    for k in range(NS):
        _router_chunk(x_refs[k][0], w, p_ref, ew_ref, ei_ref, k)


def kernel(x, W):
    xc = x.reshape(NCHUNK, CT, D_MODEL)

    def chunk_spec(k):
        return pl.BlockSpec((1, CT, D_MODEL), lambda i, k=k: (i * NS + k, 0, 0))

    probs_em, ew_em, ei_em = pl.pallas_call(
        _router_body,
        grid=(TOKENS // BT,),
        in_specs=[chunk_spec(k) for k in range(NS)]
        + [pl.BlockSpec((N_EXPERTS, D_MODEL), lambda i: (0, 0))],
        out_specs=[
            pl.BlockSpec((N_EXPERTS, BT), lambda i: (0, i)),
            pl.BlockSpec((TOP_K, BT), lambda i: (0, i)),
            pl.BlockSpec((TOP_K, BT), lambda i: (0, i)),
        ],
        out_shape=[
            jax.ShapeDtypeStruct((N_EXPERTS, TOKENS), jnp.float32),
            jax.ShapeDtypeStruct((TOP_K, TOKENS), jnp.float32),
            jax.ShapeDtypeStruct((TOP_K, TOKENS), jnp.int32),
        ],
        compiler_params=pltpu.CompilerParams(
            dimension_semantics=("arbitrary",),
        ),
    )(*([xc] * NS), W)
    return (probs_em.T, ew_em.T, ei_em.T)


# speedup vs baseline: 1.0328x; 1.0057x over previous
import jax
import jax.numpy as jnp
from jax.experimental import pallas as pl
from jax.experimental.pallas import tpu as pltpu

TOKENS = 32768
D_MODEL = 768
N_EXPERTS = 8
TOP_K = 2

CT = 1024
NS = 4
BT = CT * NS
NCHUNK = TOKENS // CT


def _router_chunk(x, w, p_ref, ew_ref, ei_ref, k):
    lt = jax.lax.dot_general(
        w, x, (((1,), (1,)), ((), ())), preferred_element_type=jnp.float32
    )
    m = jnp.max(lt, axis=0, keepdims=True)
    e = jnp.exp(lt - m)
    p = e / jnp.sum(e, axis=0, keepdims=True)
    p_ref[:, pl.ds(k * CT, CT)] = p

    m1 = p[0:1, :]
    m2 = jnp.full((1, CT), -1.0, jnp.float32)
    i1 = jnp.zeros((1, CT), jnp.int32)
    i2 = jnp.zeros((1, CT), jnp.int32)
    for ei in range(1, N_EXPERTS):
        v = p[ei : ei + 1, :]
        ec = jnp.full((1, CT), ei, jnp.int32)
        gt1 = v > m1
        gt2 = v > m2
        i2 = jnp.where(gt1, i1, jnp.where(gt2, ec, i2))
        m2 = jnp.where(gt1, m1, jnp.where(gt2, v, m2))
        i1 = jnp.where(gt1, ec, i1)
        m1 = jnp.where(gt1, v, m1)
    ew_ref[:, pl.ds(k * CT, CT)] = jnp.concatenate([m1, m2], axis=0)
    ei_ref[:, pl.ds(k * CT, CT)] = jnp.concatenate([i1, i2], axis=0)


def _router_body(*refs):
    x_refs = refs[:NS]
    w_ref, p_ref, ew_ref, ei_ref = refs[NS:]
    w = w_ref[...]
    for k in range(NS):
        _router_chunk(x_refs[k][0], w, p_ref, ew_ref, ei_ref, k)


def kernel(x, W):
    xc = x.reshape(NCHUNK, CT, D_MODEL)

    def chunk_spec(k):
        return pl.BlockSpec((1, CT, D_MODEL), lambda i, k=k: (i * NS + k, 0, 0))

    probs_em, ew_em, ei_em = pl.pallas_call(
        _router_body,
        grid=(TOKENS // BT,),
        in_specs=[chunk_spec(k) for k in range(NS)]
        + [pl.BlockSpec((N_EXPERTS, D_MODEL), lambda i: (0, 0))],
        out_specs=[
            pl.BlockSpec((N_EXPERTS, BT), lambda i: (0, i)),
            pl.BlockSpec((TOP_K, BT), lambda i: (0, i)),
            pl.BlockSpec((TOP_K, BT), lambda i: (0, i)),
        ],
        out_shape=[
            jax.ShapeDtypeStruct((N_EXPERTS, TOKENS), jnp.float32),
            jax.ShapeDtypeStruct((TOP_K, TOKENS), jnp.float32),
            jax.ShapeDtypeStruct((TOP_K, TOKENS), jnp.int32),
        ],
        compiler_params=pltpu.CompilerParams(
            dimension_semantics=("arbitrary",),
        ),
    )(*([xc] * NS), W)
    return (probs_em.T, ew_em.T, ei_em.T)
